# Initial kernel scaffold; baseline (speedup 1.0000x reference)
import jax
import jax.numpy as jnp
from jax.experimental import pallas as pl


def kernel(seq_time, time_day, time_week):
    raise NotImplementedError("write your pallas kernel here")


# speedup vs baseline: 2.3856x; 2.3856x over previous
import jax
import jax.numpy as jnp
from jax.experimental import pallas as pl

TIME = 288
FEATURES = 64
PPH = 12
NUM_NODES = 883
B = 64
T = 12
N_BLK = 128


def _emb_kernel(st_ref, td_ref, tw_ref, out_ref):
    st = st_ref[0]
    hour = (st[3:4, :] + 0.5) * 23
    minute = (st[4:5, :] + 0.5) * 59
    hour_index = ((hour * 60 + minute) / (60.0 / PPH)).astype(jnp.int32)
    day = ((st[2:3, :] + 0.5) * 6).astype(jnp.int32)

    iota_d = jax.lax.broadcasted_iota(jnp.int32, (TIME, T), 0)
    oh_d = (iota_d == hour_index).astype(jnp.float32)
    iota_w = jax.lax.broadcasted_iota(jnp.int32, (7, T), 0)
    oh_w = (iota_w == day).astype(jnp.float32)

    eT = jax.lax.dot_general(
        td_ref[...], oh_d, (((0,), (0,)), ((), ())),
        preferred_element_type=jnp.float32)
    eT = eT + jax.lax.dot_general(
        tw_ref[...], oh_w, (((0,), (0,)), ((), ())),
        preferred_element_type=jnp.float32)

    out_ref[0] = jnp.broadcast_to(eT[:, None, :], (FEATURES, N_BLK, T))


def kernel(seq_time, time_day, time_week):
    n_tiles = pl.cdiv(NUM_NODES, N_BLK)
    return pl.pallas_call(
        _emb_kernel,
        grid=(B, n_tiles),
        in_specs=[
            pl.BlockSpec((1, 5, T), lambda b, j: (b, 0, 0)),
            pl.BlockSpec((TIME, FEATURES), lambda b, j: (0, 0)),
            pl.BlockSpec((7, FEATURES), lambda b, j: (0, 0)),
        ],
        out_specs=pl.BlockSpec((1, FEATURES, N_BLK, T), lambda b, j: (b, 0, j, 0)),
        out_shape=jax.ShapeDtypeStruct((B, FEATURES, NUM_NODES, T), jnp.float32),
    )(seq_time, time_day, time_week)


# speedup vs baseline: 5.1022x; 2.1387x over previous
import jax
import jax.numpy as jnp
from jax.experimental import pallas as pl

TIME = 288
FEATURES = 64
PPH = 12
NUM_NODES = 883
B = 64
T = 12
NT = NUM_NODES * T
L_BLK = 1536


def _emb_kernel(st_ref, td_ref, tw_ref, out_ref):
    st = st_ref[0]
    hour = (st[3:4, :] + 0.5) * 23
    minute = (st[4:5, :] + 0.5) * 59
    hour_index = ((hour * 60 + minute) / (60.0 / PPH)).astype(jnp.int32)
    day = ((st[2:3, :] + 0.5) * 6).astype(jnp.int32)

    iota_d = jax.lax.broadcasted_iota(jnp.int32, (TIME, T), 0)
    oh_d = (iota_d == hour_index).astype(jnp.float32)
    iota_w = jax.lax.broadcasted_iota(jnp.int32, (7, T), 0)
    oh_w = (iota_w == day).astype(jnp.float32)

    eT = jax.lax.dot_general(
        td_ref[...], oh_d, (((0,), (0,)), ((), ())),
        preferred_element_type=jnp.float32)
    eT = eT + jax.lax.dot_general(
        tw_ref[...], oh_w, (((0,), (0,)), ((), ())),
        preferred_element_type=jnp.float32)

    lane_mod = jax.lax.broadcasted_iota(jnp.int32, (T, L_BLK), 1) % T
    row_id = jax.lax.broadcasted_iota(jnp.int32, (T, L_BLK), 0)
    sel = (lane_mod == row_id).astype(jnp.float32)
    out_ref[0] = jnp.dot(eT, sel, preferred_element_type=jnp.float32)


def kernel(seq_time, time_day, time_week):
    n_tiles = pl.cdiv(NT, L_BLK)
    flat = pl.pallas_call(
        _emb_kernel,
        grid=(B, n_tiles),
        in_specs=[
            pl.BlockSpec((1, 5, T), lambda b, j: (b, 0, 0)),
            pl.BlockSpec((TIME, FEATURES), lambda b, j: (0, 0)),
            pl.BlockSpec((7, FEATURES), lambda b, j: (0, 0)),
        ],
        out_specs=pl.BlockSpec((1, FEATURES, L_BLK), lambda b, j: (b, 0, j)),
        out_shape=jax.ShapeDtypeStruct((B, FEATURES, NT), jnp.float32),
    )(seq_time, time_day, time_week)
    return flat.reshape(B, FEATURES, NUM_NODES, T)


# speedup vs baseline: 7.0083x; 1.3736x over previous
import jax
import jax.numpy as jnp
from jax.experimental import pallas as pl

TIME = 288
FEATURES = 64
PPH = 12
NUM_NODES = 883
B = 64
T = 12
NT = NUM_NODES * T
L_BLK = 10752


def _emb_kernel(st_ref, td_ref, tw_ref, out_ref):
    st = st_ref[0]
    hour = (st[3:4, :] + 0.5) * 23
    minute = (st[4:5, :] + 0.5) * 59
    hour_index = ((hour * 60 + minute) / (60.0 / PPH)).astype(jnp.int32)
    day = ((st[2:3, :] + 0.5) * 6).astype(jnp.int32)

    iota_d = jax.lax.broadcasted_iota(jnp.int32, (TIME, T), 0)
    oh_d = (iota_d == hour_index).astype(jnp.float32)
    iota_w = jax.lax.broadcasted_iota(jnp.int32, (7, T), 0)
    oh_w = (iota_w == day).astype(jnp.float32)

    eT = jax.lax.dot_general(
        td_ref[...], oh_d, (((0,), (0,)), ((), ())),
        preferred_element_type=jnp.float32)
    eT = eT + jax.lax.dot_general(
        tw_ref[...], oh_w, (((0,), (0,)), ((), ())),
        preferred_element_type=jnp.float32)

    lane_mod = jax.lax.broadcasted_iota(jnp.int32, (T, L_BLK), 1) % T
    row_id = jax.lax.broadcasted_iota(jnp.int32, (T, L_BLK), 0)
    sel = (lane_mod == row_id).astype(jnp.float32)
    out_ref[0] = jnp.dot(eT, sel, preferred_element_type=jnp.float32)


def kernel(seq_time, time_day, time_week):
    n_tiles = pl.cdiv(NT, L_BLK)
    flat = pl.pallas_call(
        _emb_kernel,
        grid=(B, n_tiles),
        in_specs=[
            pl.BlockSpec((1, 5, T), lambda b, j: (b, 0, 0)),
            pl.BlockSpec((TIME, FEATURES), lambda b, j: (0, 0)),
            pl.BlockSpec((7, FEATURES), lambda b, j: (0, 0)),
        ],
        out_specs=pl.BlockSpec((1, FEATURES, L_BLK), lambda b, j: (b, 0, j)),
        out_shape=jax.ShapeDtypeStruct((B, FEATURES, NT), jnp.float32),
    )(seq_time, time_day, time_week)
    return flat.reshape(B, FEATURES, NUM_NODES, T)
